# Initial kernel scaffold; baseline (speedup 1.0000x reference)
import jax
import jax.numpy as jnp
from jax.experimental import pallas as pl


def kernel(voxel, fish_cell_indices, cell_indices):
    raise NotImplementedError("write your pallas kernel here")


# speedup vs baseline: 3.2888x; 3.2888x over previous
import functools

import jax
import jax.numpy as jnp
from jax import lax
from jax.experimental import pallas as pl
from jax.experimental.pallas import tpu as pltpu
from jax.experimental.pallas import tpu_sc as plsc

_N = 2097152
_NIDX = 262144
_EPS = 1e-07
_ONE_PLUS_EPS = float(jnp.float32(1.0) + jnp.float32(_EPS))

_NC = 2
_NS = 16
_NW = _NC * _NS
_IDX_PER_W = _NIDX // _NW
_CHUNK = 128
_NCHUNK = _IDX_PER_W // _CHUNK
_FIRE = 8


def _fill_body(o_ref):
    o_ref[...] = jnp.full(o_ref.shape, _EPS, jnp.float32)


_fill = pl.pallas_call(
    _fill_body,
    out_shape=jax.ShapeDtypeStruct((2048, 1024), jnp.float32),
    grid=(8,),
    out_specs=pl.BlockSpec((256, 1024), lambda i: (i, 0)),
)


_mesh = plsc.VectorSubcoreMesh(core_axis_name="c", subcore_axis_name="s")


@functools.partial(
    pl.kernel,
    mesh=_mesh,
    scratch_types=[
        pltpu.VMEM((_NCHUNK, _CHUNK), jnp.int32),
        pltpu.VMEM((_CHUNK,), jnp.float32),
        pltpu.SemaphoreType.DMA,
    ],
)
def _scatter(idx_hbm, out_ref, idx_v, vals_v, sem):
    wid = lax.axis_index("s") * _NC + lax.axis_index("c")
    pltpu.sync_copy(idx_hbm.at[pl.ds(wid * _NCHUNK, _NCHUNK)], idx_v)
    one = jnp.full((16,), _ONE_PLUS_EPS, jnp.float32)
    for i in range(_CHUNK // 16):
        vals_v[pl.ds(i * 16, 16)] = one

    @pl.loop(0, _NCHUNK // _FIRE)
    def _group(g):
        base = g * _FIRE
        for b in range(_FIRE):
            pltpu.make_async_copy(
                vals_v, out_ref.at[idx_v.at[base + b]], sem
            ).start()
        for b in range(_FIRE):
            pltpu.make_async_copy(
                vals_v, out_ref.at[idx_v.at[base + b]], sem
            ).wait()


def kernel(voxel, fish_cell_indices, cell_indices):
    del voxel, cell_indices
    filled = _fill().reshape(_N)
    idx2d = fish_cell_indices.reshape(_NW * _NCHUNK, _CHUNK)
    out_ref = jax.new_ref(filled)
    _scatter(idx2d, out_ref)
    return jax.freeze(out_ref)


# speedup vs baseline: 3.2891x; 1.0001x over previous
import functools

import jax
import jax.numpy as jnp
from jax import lax
from jax.experimental import pallas as pl
from jax.experimental.pallas import tpu as pltpu
from jax.experimental.pallas import tpu_sc as plsc

_N = 2097152
_NIDX = 262144
_EPS = 1e-07
_ONE_PLUS_EPS = float(jnp.float32(1.0) + jnp.float32(_EPS))

_NC = 2
_NS = 16
_NW = _NC * _NS
_IDX_PER_W = _NIDX // _NW
_CHUNK = 128
_NCHUNK = _IDX_PER_W // _CHUNK
_FIRE = 8


def _fill_body(o_ref):
    o_ref[...] = jnp.full(o_ref.shape, _EPS, jnp.float32)


_fill = pl.pallas_call(
    _fill_body,
    out_shape=jax.ShapeDtypeStruct((2048, 1024), jnp.float32),
    grid=(8,),
    out_specs=pl.BlockSpec((256, 1024), lambda i: (i, 0)),
)


_mesh = plsc.VectorSubcoreMesh(core_axis_name="c", subcore_axis_name="s")


@functools.partial(
    pl.kernel,
    mesh=_mesh,
    scratch_types=[
        pltpu.VMEM((_IDX_PER_W,), jnp.int32),
        pltpu.VMEM((_IDX_PER_W,), jnp.float32),
        pltpu.SemaphoreType.DMA,
    ],
)
def _scatter(idx_hbm, out_ref, idx_v, vals_v, sem):
    wid = lax.axis_index("s") * _NC + lax.axis_index("c")
    pltpu.sync_copy(idx_hbm.at[pl.ds(wid * _IDX_PER_W, _IDX_PER_W)], idx_v)
    one = jnp.full((16,), _ONE_PLUS_EPS, jnp.float32)

    @pl.loop(0, _IDX_PER_W // 16, unroll=8)
    def _fill_vals(i):
        vals_v[pl.ds(i * 16, 16)] = one

    pltpu.async_copy(vals_v, out_ref.at[idx_v], sem).wait()


def kernel(voxel, fish_cell_indices, cell_indices):
    del voxel, cell_indices
    filled = _fill().reshape(_N)
    out_ref = jax.new_ref(filled)
    _scatter(fish_cell_indices, out_ref)
    return jax.freeze(out_ref)


# speedup vs baseline: 9.3357x; 2.8383x over previous
import functools

import jax
import jax.numpy as jnp
import numpy as np
from jax import lax
from jax.experimental import pallas as pl
from jax.experimental.pallas import tpu as pltpu
from jax.experimental.pallas import tpu_sc as plsc

_N = 2097152
_NIDX = 262144
_EPS = 1e-07
_ONE_PLUS_EPS = float(np.float32(1.0) + np.float32(_EPS))

_NC = 2
_NS = 16
_HALF = _N // _NC
_DUMMY = _HALF
_IDXC = _NIDX // _NS
_SLICE = _HALF // _NS

_mesh = plsc.VectorSubcoreMesh(core_axis_name="c", subcore_axis_name="s")


@functools.partial(
    pl.kernel,
    mesh=_mesh,
    out_type=jax.ShapeDtypeStruct((_N,), jnp.float32),
    scratch_types=[
        pltpu.VMEM((_IDXC,), jnp.int32),
        pltpu.VMEM((_IDXC,), jnp.float32),
        pltpu.VMEM((_IDXC,), jnp.float32),
        pltpu.VMEM_SHARED((_HALF + 8,), jnp.float32),
        pltpu.SemaphoreType.DMA,
        pltpu.SemaphoreType.DMA,
    ],
)
def _voxel_scatter(idx_hbm, out_hbm, idx_v, vals_v, eps_v, spmem, sem_a, sem_b):
    c = lax.axis_index("c")
    s = lax.axis_index("s")
    base = c * _HALF

    idx_load = pltpu.make_async_copy(
        idx_hbm.at[pl.ds(s * _IDXC, _IDXC)], idx_v, sem_a
    )
    idx_load.start()

    eps16 = jnp.full((16,), _EPS, jnp.float32)
    one16 = jnp.full((16,), _ONE_PLUS_EPS, jnp.float32)

    @pl.loop(0, _IDXC // 16, unroll=8)
    def _fill_eps(i):
        eps_v[pl.ds(i * 16, 16)] = eps16

    @pl.loop(0, _IDXC // 16, unroll=8)
    def _fill_vals(i):
        vals_v[pl.ds(i * 16, 16)] = one16

    inits = []
    for k in range(_SLICE // _IDXC):
        cp = pltpu.make_async_copy(
            eps_v, spmem.at[pl.ds(s * _SLICE + k * _IDXC, _IDXC)], sem_b
        )
        cp.start()
        inits.append(cp)

    idx_load.wait()
    dummy16 = jnp.full((16,), _DUMMY, jnp.int32)

    @pl.loop(0, _IDXC // 16, unroll=4)
    def _rebase(i):
        v = idx_v[pl.ds(i * 16, 16)]
        local = v - base
        m = (v >= base) & (local < _HALF)
        idx_v[pl.ds(i * 16, 16)] = jnp.where(m, local, dummy16)

    for cp in inits:
        cp.wait()
    plsc.subcore_barrier()

    pltpu.async_copy(vals_v, spmem.at[idx_v], sem_a).wait()
    plsc.subcore_barrier()

    pltpu.sync_copy(
        spmem.at[pl.ds(s * _SLICE, _SLICE)],
        out_hbm.at[pl.ds(base + s * _SLICE, _SLICE)],
    )


def kernel(voxel, fish_cell_indices, cell_indices):
    del voxel, cell_indices
    return _voxel_scatter(fish_cell_indices)
